# baseline (speedup 1.0000x reference)
import functools

import jax
import jax.numpy as jnp
from jax import lax
from jax.experimental import pallas as pl
from jax.experimental.pallas import tpu as pltpu
from jax.experimental.pallas import tpu_sc as plsc

B = 32
D = 64
T = 1024
K = 1024
_LANES = 16
_HALF = T // 2
_PB = 2


def _tc_body(x_ref, emb_ref, idx_ref, lsum_ref, embt_ref):
    step = pl.program_id(0)
    emb = emb_ref[...]
    emb2 = emb + emb
    en2 = jnp.sum(emb * emb, axis=1)

    @pl.when(step == 0)
    def _():
        embt_ref[...] = emb.T

    acc = jnp.float32(0.0)
    for j in range(_PB):
        xb = x_ref[j]
        mm2 = lax.dot_general(emb2, xb, (((1,), (0,)), ((), ())),
                              preferred_element_type=jnp.float32)
        xn2 = jnp.sum(xb * xb, axis=0)
        dist = (xn2[None, :] + en2[:, None]) - mm2
        m = jnp.min(dist, axis=0)
        hit = dist == m[None, :]
        iota = lax.broadcasted_iota(jnp.int32, (K, T), 0)
        idxs = jnp.min(jnp.where(hit, iota, K), axis=0)
        idx_ref[j, 0, :] = idxs
        acc = acc + jnp.sum(m)
    prev = jnp.where(step == 0, 0.0, lsum_ref[0, 0])
    lsum_ref[0, 0] = prev + acc


_tc_call = pl.pallas_call(
    _tc_body,
    grid=(B // _PB,),
    in_specs=[
        pl.BlockSpec((_PB, D, T), lambda i: (i, 0, 0)),
        pl.BlockSpec((K, D), lambda i: (0, 0)),
    ],
    out_specs=[
        pl.BlockSpec((_PB, 1, T), lambda i: (i, 0, 0)),
        pl.BlockSpec((1, 1), lambda i: (0, 0), memory_space=pltpu.SMEM),
        pl.BlockSpec((D, K), lambda i: (0, 0)),
    ],
    out_shape=[
        jax.ShapeDtypeStruct((B, 1, T), jnp.int32),
        jax.ShapeDtypeStruct((1, 1), jnp.float32),
        jax.ShapeDtypeStruct((D, K), jnp.float32),
    ],
)


@functools.cache
def _make_sc_gather():
    mesh = plsc.VectorSubcoreMesh(core_axis_name="c", subcore_axis_name="s")

    @functools.partial(
        pl.kernel,
        mesh=mesh,
        out_type=jax.ShapeDtypeStruct((B, D, T), jnp.float32),
        compiler_params=pltpu.CompilerParams(needs_layout_passes=False),
        scratch_types=[
            pltpu.VMEM((D * K,), jnp.float32),
            pltpu.VMEM((T,), jnp.int32),
            pltpu.VMEM((D, _HALF), jnp.float32),
        ],
    )
    def _sc_gather(idx_hbm, embt_hbm, out_hbm, emb_v, idx_v, out_v):
        c = lax.axis_index("c")
        s = lax.axis_index("s")
        b = s * 2 + c
        pltpu.sync_copy(embt_hbm, emb_v)
        pltpu.sync_copy(idx_hbm.at[b], idx_v)
        for h in range(2):
            @plsc.parallel_loop(0, _HALF // _LANES, unroll=2)
            def body(g):
                tb = h * _HALF + g * _LANES
                idxv = idx_v[pl.ds(tb, _LANES)]
                for d in range(D):
                    val = plsc.load_gather(emb_v, [idxv + d * K])
                    out_v[d, pl.ds(g * _LANES, _LANES)] = val

            pltpu.sync_copy(out_v, out_hbm.at[b, :, pl.ds(h * _HALF, _HALF)])

    return _sc_gather


def kernel(x, embedding):
    idx3, lsum, embt = _tc_call(x, embedding)
    indexes = jnp.reshape(idx3, (B, T))
    values = _make_sc_gather()(indexes, jnp.reshape(embt, (D * K,)))
    loss = jnp.reshape(lsum, ()) * (2.0 / (B * T * D))
    return (values, indexes, loss)


# speedup vs baseline: 1.0983x; 1.0983x over previous
import functools

import jax
import jax.numpy as jnp
from jax import lax
from jax.experimental import pallas as pl
from jax.experimental.pallas import tpu as pltpu
from jax.experimental.pallas import tpu_sc as plsc

B = 32
D = 64
T = 1024
K = 1024
_LANES = 16
_HALF = T // 2
_PB = 4


def _tc_body(x_ref, emb_ref, idx_ref, lsum_ref):
    step = pl.program_id(0)
    emb = emb_ref[...]
    emb2 = emb + emb
    en2 = jnp.sum(emb * emb, axis=1)
    acc = jnp.float32(0.0)
    for j in range(_PB):
        xb = x_ref[j]
        mm2 = lax.dot_general(emb2, xb, (((1,), (0,)), ((), ())),
                              preferred_element_type=jnp.float32)
        xn2 = jnp.sum(xb * xb, axis=0)
        dist = (xn2[None, :] + en2[:, None]) - mm2
        m = jnp.min(dist, axis=0)
        hit = dist == m[None, :]
        iota = lax.broadcasted_iota(jnp.int32, (K, T), 0)
        idxs = jnp.min(jnp.where(hit, iota, K), axis=0)
        idx_ref[j, 0, :] = idxs
        acc = acc + jnp.sum(m)
    prev = jnp.where(step == 0, 0.0, lsum_ref[0, 0])
    lsum_ref[0, 0] = prev + acc


_tc_call = pl.pallas_call(
    _tc_body,
    grid=(B // _PB,),
    in_specs=[
        pl.BlockSpec((_PB, D, T), lambda i: (i, 0, 0)),
        pl.BlockSpec((K, D), lambda i: (0, 0)),
    ],
    out_specs=[
        pl.BlockSpec((_PB, 1, T), lambda i: (i, 0, 0)),
        pl.BlockSpec((1, 1), lambda i: (0, 0), memory_space=pltpu.SMEM),
    ],
    out_shape=[
        jax.ShapeDtypeStruct((B, 1, T), jnp.int32),
        jax.ShapeDtypeStruct((1, 1), jnp.float32),
    ],
)


@functools.cache
def _make_sc_gather():
    mesh = plsc.VectorSubcoreMesh(core_axis_name="c", subcore_axis_name="s")

    @functools.partial(
        pl.kernel,
        mesh=mesh,
        out_type=jax.ShapeDtypeStruct((B, D, T), jnp.float32),
        compiler_params=pltpu.CompilerParams(needs_layout_passes=False),
        scratch_types=[
            pltpu.VMEM((D * K,), jnp.float32),
            pltpu.VMEM((T,), jnp.int32),
            pltpu.VMEM((D, _HALF), jnp.float32),
        ],
    )
    def _sc_gather(idx_hbm, embt_hbm, out_hbm, emb_v, idx_v, out_v):
        c = lax.axis_index("c")
        s = lax.axis_index("s")
        b = s * 2 + c
        pltpu.sync_copy(embt_hbm, emb_v)
        pltpu.sync_copy(idx_hbm.at[b, 0], idx_v)
        for h in range(2):
            @plsc.parallel_loop(0, _HALF // _LANES, unroll=2)
            def body(g):
                tb = h * _HALF + g * _LANES
                idxv = idx_v[pl.ds(tb, _LANES)]
                for d in range(D):
                    val = plsc.load_gather(emb_v, [idxv + d * K])
                    out_v[d, pl.ds(g * _LANES, _LANES)] = val

            pltpu.sync_copy(out_v, out_hbm.at[b, :, pl.ds(h * _HALF, _HALF)])

    return _sc_gather


def kernel(x, embedding):
    embt = jnp.reshape(jnp.transpose(embedding), (D * K,))
    idx3, lsum = _tc_call(x, embedding)
    indexes = jnp.reshape(idx3, (B, T))
    values = _make_sc_gather()(idx3, embt)
    loss = jnp.reshape(lsum, ()) * (2.0 / (B * T * D))
    return (values, indexes, loss)


# speedup vs baseline: 1.1011x; 1.0026x over previous
import functools

import jax
import jax.numpy as jnp
from jax import lax
from jax.experimental import pallas as pl
from jax.experimental.pallas import tpu as pltpu
from jax.experimental.pallas import tpu_sc as plsc

B = 32
D = 64
T = 1024
K = 1024
_LANES = 16
_HALF = T // 2
_PB = 8


def _tc_body(x_ref, emb_ref, idx_ref, lsum_ref):
    step = pl.program_id(0)
    emb = emb_ref[...]
    emb2 = emb + emb
    en2 = jnp.sum(emb * emb, axis=1)
    acc = jnp.float32(0.0)
    for j in range(_PB):
        xb = x_ref[j]
        mm2 = lax.dot_general(emb2, xb, (((1,), (0,)), ((), ())),
                              preferred_element_type=jnp.float32)
        xn2 = jnp.sum(xb * xb, axis=0)
        dist = (xn2[None, :] + en2[:, None]) - mm2
        m = jnp.min(dist, axis=0)
        hit = dist == m[None, :]
        iota = lax.broadcasted_iota(jnp.int32, (K, T), 0)
        idxs = jnp.min(jnp.where(hit, iota, K), axis=0)
        idx_ref[j, 0, :] = idxs
        acc = acc + jnp.sum(m)
    prev = jnp.where(step == 0, 0.0, lsum_ref[0, 0])
    lsum_ref[0, 0] = prev + acc


_tc_call = pl.pallas_call(
    _tc_body,
    grid=(B // _PB,),
    in_specs=[
        pl.BlockSpec((_PB, D, T), lambda i: (i, 0, 0)),
        pl.BlockSpec((K, D), lambda i: (0, 0)),
    ],
    out_specs=[
        pl.BlockSpec((_PB, 1, T), lambda i: (i, 0, 0)),
        pl.BlockSpec((1, 1), lambda i: (0, 0), memory_space=pltpu.SMEM),
    ],
    out_shape=[
        jax.ShapeDtypeStruct((B, 1, T), jnp.int32),
        jax.ShapeDtypeStruct((1, 1), jnp.float32),
    ],
)


@functools.cache
def _make_sc_gather():
    mesh = plsc.VectorSubcoreMesh(core_axis_name="c", subcore_axis_name="s")

    @functools.partial(
        pl.kernel,
        mesh=mesh,
        out_type=jax.ShapeDtypeStruct((B, D, T), jnp.float32),
        compiler_params=pltpu.CompilerParams(needs_layout_passes=False),
        scratch_types=[
            pltpu.VMEM((D * K,), jnp.float32),
            pltpu.VMEM((T,), jnp.int32),
            pltpu.VMEM((D, _HALF), jnp.float32),
        ],
    )
    def _sc_gather(idx_hbm, embt_hbm, out_hbm, emb_v, idx_v, out_v):
        c = lax.axis_index("c")
        s = lax.axis_index("s")
        b = s * 2 + c
        pltpu.sync_copy(embt_hbm, emb_v)
        pltpu.sync_copy(idx_hbm.at[b, 0], idx_v)
        for h in range(2):
            @plsc.parallel_loop(0, _HALF // _LANES, unroll=2)
            def body(g):
                tb = h * _HALF + g * _LANES
                idxv = idx_v[pl.ds(tb, _LANES)]
                for d in range(D):
                    val = plsc.load_gather(emb_v, [idxv + d * K])
                    out_v[d, pl.ds(g * _LANES, _LANES)] = val

            pltpu.sync_copy(out_v, out_hbm.at[b, :, pl.ds(h * _HALF, _HALF)])

    return _sc_gather


def kernel(x, embedding):
    embt = jnp.reshape(jnp.transpose(embedding), (D * K,))
    idx3, lsum = _tc_call(x, embedding)
    indexes = jnp.reshape(idx3, (B, T))
    values = _make_sc_gather()(idx3, embt)
    loss = jnp.reshape(lsum, ()) * (2.0 / (B * T * D))
    return (values, indexes, loss)
